# baseline (speedup 1.0000x reference)
import functools

import jax
import jax.numpy as jnp
from jax import lax
from jax.experimental import pallas as pl
from jax.experimental.pallas import tpu as pltpu
from jax.experimental.pallas import tpu_sc as plsc

_DIM = 128
_MAX_LEN = 200
_LANES = 16
_NW_FULL = 12


def _gather_body(slen_hbm, emb_hbm, out_hbm, slen_v, idx_v, rows_v, sem):
    wid = lax.axis_index("s")

    @pl.when(wid <= _NW_FULL)
    def _():
        base = pl.multiple_of(wid * _LANES, 8)
        nrows = jnp.where(wid < _NW_FULL, _LANES, _MAX_LEN - _NW_FULL * _LANES)
        cp = pltpu.async_copy(emb_hbm.at[pl.ds(base, 8)],
                              out_hbm.at[pl.ds(base, 8)], sem)
        pltpu.sync_copy(slen_hbm, slen_v)
        slen_s = slen_v[...][0]

        @pl.when(wid < _NW_FULL)
        def _():
            pltpu.sync_copy(emb_hbm.at[pl.ds(base + 8, 8)],
                            out_hbm.at[pl.ds(base + 8, 8)])

        cp.wait()

        @pl.when(base + nrows - 1 > slen_s - 1)
        def _():
            lane = lax.iota(jnp.int32, _LANES)
            idx_v[...] = jnp.minimum(jnp.minimum(base + lane, _MAX_LEN - 1),
                                     slen_s - 1)
            pltpu.async_copy(emb_hbm.at[idx_v], rows_v, sem).wait()

            @pl.when(wid < _NW_FULL)
            def _():
                pltpu.sync_copy(rows_v, out_hbm.at[pl.ds(base, _LANES)])

            @pl.when(wid == _NW_FULL)
            def _():
                pltpu.sync_copy(rows_v.at[pl.ds(0, 8)],
                                out_hbm.at[pl.ds(base, 8)])


@jax.jit
def _gather(slen_vec, emb):
    mesh = plsc.VectorSubcoreMesh(core_axis_name="c", subcore_axis_name="s",
                                  num_cores=1)
    return pl.kernel(
        _gather_body,
        mesh=mesh,
        out_type=jax.ShapeDtypeStruct((_MAX_LEN, _DIM), jnp.float32),
        scratch_types=[
            pltpu.VMEM((_LANES,), jnp.int32),
            pltpu.VMEM((_LANES,), jnp.int32),
            pltpu.VMEM((_LANES, _DIM), jnp.float32),
            pltpu.SemaphoreType.DMA,
        ],
    )(slen_vec, emb)


def kernel(seq_len, emb):
    slen_vec = jnp.full((_LANES,), seq_len, dtype=jnp.int32)
    return _gather(slen_vec, emb)


# speedup vs baseline: 1.1203x; 1.1203x over previous
import jax
import jax.numpy as jnp
from jax import lax
from jax.experimental import pallas as pl
from jax.experimental.pallas import tpu as pltpu
from jax.experimental.pallas import tpu_sc as plsc

_DIM = 128
_MAX_LEN = 200
_LANES = 16
_ROWS_PER_W = 8
_NW_FULL = 12


def _gather_body(slen_hbm, emb_hbm, out_hbm, slen_v, idx_v, rows_v, sem):
    wid = lax.axis_index("s")

    @pl.when(wid <= _NW_FULL)
    def _():
        base = pl.multiple_of(wid * _LANES, _ROWS_PER_W)
        pltpu.sync_copy(slen_hbm, slen_v)
        lane = lax.iota(jnp.int32, _LANES)
        idx_v[...] = jnp.minimum(base + lane, slen_v[...] - 1)
        pltpu.async_copy(emb_hbm.at[idx_v], rows_v, sem).wait()

        @pl.when(wid < _NW_FULL)
        def _():
            pltpu.sync_copy(rows_v, out_hbm.at[pl.ds(base, _LANES)])

        @pl.when(wid == _NW_FULL)
        def _():
            pltpu.sync_copy(rows_v.at[pl.ds(0, _ROWS_PER_W)],
                            out_hbm.at[pl.ds(base, _ROWS_PER_W)])


@jax.jit
def _gather(slen_vec, emb):
    mesh = plsc.VectorSubcoreMesh(core_axis_name="c", subcore_axis_name="s",
                                  num_cores=1)
    return pl.kernel(
        _gather_body,
        mesh=mesh,
        out_type=jax.ShapeDtypeStruct((_MAX_LEN, _DIM), jnp.float32),
        scratch_types=[
            pltpu.VMEM((_LANES,), jnp.int32),
            pltpu.VMEM((_LANES,), jnp.int32),
            pltpu.VMEM((_LANES, _DIM), jnp.float32),
            pltpu.SemaphoreType.DMA,
        ],
    )(slen_vec, emb)


def kernel(seq_len, emb):
    slen_vec = jnp.full((_LANES,), seq_len, dtype=jnp.int32)
    return _gather(slen_vec, emb)
